# baseline (speedup 1.0000x reference)
import jax
import jax.numpy as jnp
from jax.experimental import pallas as pl
from jax.experimental.pallas import tpu as pltpu

_EPS = 1e-05
_OUT = 512
_KSZ = 8
_ROWS_PER_BLOCK = 256


def _lrp_pool_body(a_ref, r_ref, o_ref):
    x = a_ref[...]
    rr = r_ref[...]
    n = x.shape[1]
    acc = x
    for s in (4, 2, 1):
        acc = acc + pltpu.roll(acc, n - s, axis=1)
    li = jax.lax.broadcasted_iota(jnp.int32, (n, n), 0)
    lj = jax.lax.broadcasted_iota(jnp.int32, (n, n), 1)
    bsel = jnp.where(li == _KSZ * (lj // _KSZ), 1.0, 0.0).astype(x.dtype)
    zsum = jax.lax.dot(acc, bsel, precision=jax.lax.Precision.HIGHEST)
    z_full = zsum * (1.0 / _KSZ) + _EPS

    r_rep = jnp.repeat(rr, _KSZ, axis=0)
    row = jax.lax.broadcasted_iota(jnp.int32, x.shape, 0)
    lane = jax.lax.broadcasted_iota(jnp.int32, x.shape, 1)
    idx = 16 * (row % 8) + lane // _KSZ
    r_full = jnp.take_along_axis(r_rep, idx, axis=1)

    o_ref[...] = x * (r_full / z_full) * (1.0 / _KSZ)


def kernel(a, r):
    B, C, L = a.shape
    R = B * C
    rb = _ROWS_PER_BLOCK
    a2 = a.reshape(R * (L // 128), 128)
    r2 = r.reshape(R * (_OUT // 128), 128)
    out = pl.pallas_call(
        _lrp_pool_body,
        grid=(R // rb,),
        in_specs=[
            pl.BlockSpec((rb * (L // 128), 128), lambda i: (i, 0)),
            pl.BlockSpec((rb * (_OUT // 128), 128), lambda i: (i, 0)),
        ],
        out_specs=pl.BlockSpec((rb * (L // 128), 128), lambda i: (i, 0)),
        out_shape=jax.ShapeDtypeStruct((R * (L // 128), 128), a.dtype),
    )(a2, r2)
    return out.reshape(B, C, L)


# speedup vs baseline: 2.2669x; 2.2669x over previous
import jax
import jax.numpy as jnp
from jax.experimental import pallas as pl
from jax.experimental.pallas import tpu as pltpu

_EPS = 1e-05
_OUT = 512
_KSZ = 8
_ROWS_PER_BLOCK = 256


def _lrp_pool_body(a_ref, r_ref, o_ref):
    x = a_ref[...]
    rr = r_ref[...]
    rb, L = x.shape
    x3 = x.reshape(rb, L // 128, 128)
    acc = x3
    for s in (4, 2, 1):
        acc = acc + pltpu.roll(acc, 128 - s, axis=2)
    lane3 = jax.lax.broadcasted_iota(jnp.int32, x3.shape, 2)
    zsum = jnp.take_along_axis(acc, (lane3 // _KSZ) * _KSZ, axis=2)
    z_full = zsum.reshape(rb, L) * (1.0 / _KSZ) + _EPS

    idx = jax.lax.broadcasted_iota(jnp.int32, (rb, 128 * _KSZ), 1) // _KSZ
    parts = [
        jnp.take_along_axis(rr[:, q * 128:(q + 1) * 128], idx, axis=1)
        for q in range(_OUT // 128)
    ]
    r_full = jnp.concatenate(parts, axis=1)

    o_ref[...] = x * (r_full / z_full) * (1.0 / _KSZ)


def kernel(a, r):
    B, C, L = a.shape
    R = B * C
    rb = _ROWS_PER_BLOCK
    a2 = a.reshape(R, L)
    r2 = r.reshape(R, _OUT)
    out = pl.pallas_call(
        _lrp_pool_body,
        grid=(R // rb,),
        in_specs=[
            pl.BlockSpec((rb, L), lambda i: (i, 0)),
            pl.BlockSpec((rb, _OUT), lambda i: (i, 0)),
        ],
        out_specs=pl.BlockSpec((rb, L), lambda i: (i, 0)),
        out_shape=jax.ShapeDtypeStruct((R, L), a.dtype),
    )(a2, r2)
    return out.reshape(B, C, L)
